# baseline (speedup 1.0000x reference)
import functools

import jax
import jax.numpy as jnp
import numpy as np
from jax import lax
from jax.experimental import pallas as pl
from jax.experimental.pallas import tpu as pltpu
from jax.experimental.pallas import tpu_sc as plsc

N = 10000
E = 320000
D = 128
H = 8
C = D // H
L = 3

NC = 2
NS = 16
NW = NC * NS
EW = E // NW
CH = 40
NJ = EW // CH
NP = 10240
NPS = NP // NS

_BM = 1000
_BE = 4000

_mesh = plsc.VectorSubcoreMesh(core_axis_name="c", subcore_axis_name="s")


def _sc_gather(q2, kv2, dst3, src3):
    @functools.partial(
        pl.kernel,
        out_type=[
            jax.ShapeDtypeStruct((E, D), jnp.float32),
            jax.ShapeDtypeStruct((E, 2 * D), jnp.float32),
        ],
        mesh=_mesh,
        scratch_types=[
            pltpu.VMEM((NJ, CH), jnp.int32),
            pltpu.VMEM((NJ, CH), jnp.int32),
            pltpu.VMEM((CH, D), jnp.float32),
            pltpu.VMEM((CH, D), jnp.float32),
            pltpu.VMEM((CH, 2 * D), jnp.float32),
            pltpu.VMEM((CH, 2 * D), jnp.float32),
            pltpu.SemaphoreType.DMA((2,)),
            pltpu.SemaphoreType.DMA((2,)),
        ],
    )
    def body(q_hbm, kv_hbm, dst_hbm, src_hbm, qd_hbm, kvs_hbm,
             idxd, idxs, bq0, bq1, bkv0, bkv1, gsem, wsem):
        cid = lax.axis_index("c")
        sid = lax.axis_index("s")
        wid = cid * NS + sid
        base = wid * EW
        pltpu.sync_copy(dst_hbm.at[wid], idxd)
        pltpu.sync_copy(src_hbm.at[wid], idxs)
        bq = (bq0, bq1)
        bkv = (bkv0, bkv1)

        def gs(b, j):
            pltpu.async_copy(q_hbm.at[idxd.at[j]], bq[b], gsem.at[b])
            pltpu.async_copy(kv_hbm.at[idxs.at[j]], bkv[b], gsem.at[b])

        def gw(b, j):
            pltpu.make_async_copy(q_hbm.at[idxd.at[j]], bq[b],
                                  gsem.at[b]).wait()
            pltpu.make_async_copy(kv_hbm.at[idxs.at[j]], bkv[b],
                                  gsem.at[b]).wait()

        def ws(b, j):
            o = base + j * CH
            pltpu.async_copy(bq[b], qd_hbm.at[pl.ds(o, CH)], wsem.at[b])
            pltpu.async_copy(bkv[b], kvs_hbm.at[pl.ds(o, CH)], wsem.at[b])

        def ww(b, j):
            o = base + j * CH
            pltpu.make_async_copy(bq[b], qd_hbm.at[pl.ds(o, CH)],
                                  wsem.at[b]).wait()
            pltpu.make_async_copy(bkv[b], kvs_hbm.at[pl.ds(o, CH)],
                                  wsem.at[b]).wait()

        gs(0, 0)
        gs(1, 1)
        gw(0, 0)
        ws(0, 0)

        @pl.loop(0, (NJ - 2) // 2)
        def _(jj):
            p = 2 + 2 * jj
            ww(0, p - 2)
            gs(0, p)
            gw(1, p - 1)
            ws(1, p - 1)
            ww(1, p - 1)
            gs(1, p + 1)
            gw(0, p)
            ws(0, p)

        ww(0, NJ - 2)
        gw(1, NJ - 1)
        ws(1, NJ - 1)
        ww(1, NJ - 1)

    return body(q2, kv2, dst3, src3)


def _sc_scat(ed, dst3, zrow):
    ew = ed.shape[0] // NW
    nj = ew // CH
    wdt = ed.shape[1]

    @functools.partial(
        pl.kernel,
        out_type=jax.ShapeDtypeStruct((NC, NP, wdt), jnp.float32),
        mesh=_mesh,
        scratch_types=[
            pltpu.VMEM_SHARED((NP, wdt), jnp.float32),
            pltpu.VMEM((nj, CH), jnp.int32),
            pltpu.VMEM((CH, wdt), jnp.float32),
        ],
    )
    def body(ed_hbm, dst_hbm, z_hbm, acc_hbm, acc, idx, buf):
        cid = lax.axis_index("c")
        sid = lax.axis_index("s")
        wid = cid * NS + sid
        base = wid * ew
        stripe = pl.ds(sid * NPS, NPS)
        pltpu.sync_copy(z_hbm, acc.at[stripe])
        pltpu.sync_copy(dst_hbm.at[wid], idx)
        plsc.subcore_barrier()

        @pl.loop(0, nj)
        def _(j):
            pltpu.sync_copy(ed_hbm.at[pl.ds(base + j * CH, CH)], buf)
            pltpu.sync_copy(buf, acc.at[idx.at[j]], add=True)

        plsc.subcore_barrier()
        pltpu.sync_copy(acc.at[stripe], acc_hbm.at[cid].at[stripe])

    return body(ed, dst3, zrow)


def _linear_body(h_ref, w_ref, b_ref, o_ref):
    o_ref[...] = (
        jnp.dot(h_ref[...], w_ref[...], preferred_element_type=jnp.float32)
        + b_ref[...]
    )


def _linear(h, W, b):
    m, kdim = h.shape
    n = W.shape[1]
    return pl.pallas_call(
        _linear_body,
        grid=(m // _BM,),
        in_specs=[
            pl.BlockSpec((_BM, kdim), lambda i: (i, 0)),
            pl.BlockSpec((kdim, n), lambda i: (0, 0)),
            pl.BlockSpec((1, n), lambda i: (0, 0)),
        ],
        out_specs=pl.BlockSpec((_BM, n), lambda i: (i, 0)),
        out_shape=jax.ShapeDtypeStruct((m, n), jnp.float32),
    )(h, W, b.reshape(1, n))


def _proj3_body(h_ref, wq_ref, bq_ref, wkv_ref, bkv_ref, ws_ref, bs_ref,
                q_ref, kv_ref, s_ref):
    h = h_ref[...]
    q_ref[...] = (
        jnp.dot(h, wq_ref[...], preferred_element_type=jnp.float32)
        + bq_ref[...]
    )
    kv_ref[...] = (
        jnp.dot(h, wkv_ref[...], preferred_element_type=jnp.float32)
        + bkv_ref[...]
    )
    s_ref[...] = (
        jnp.dot(h, ws_ref[...], preferred_element_type=jnp.float32)
        + bs_ref[...]
    )


def _proj3(h, Wq, bq, Wkv, bkv, Ws, bs):
    return pl.pallas_call(
        _proj3_body,
        grid=(N // _BM,),
        in_specs=[
            pl.BlockSpec((_BM, D), lambda i: (i, 0)),
            pl.BlockSpec((D, D), lambda i: (0, 0)),
            pl.BlockSpec((1, D), lambda i: (0, 0)),
            pl.BlockSpec((D, 2 * D), lambda i: (0, 0)),
            pl.BlockSpec((1, 2 * D), lambda i: (0, 0)),
            pl.BlockSpec((D, D), lambda i: (0, 0)),
            pl.BlockSpec((1, D), lambda i: (0, 0)),
        ],
        out_specs=[
            pl.BlockSpec((_BM, D), lambda i: (i, 0)),
            pl.BlockSpec((_BM, 2 * D), lambda i: (i, 0)),
            pl.BlockSpec((_BM, D), lambda i: (i, 0)),
        ],
        out_shape=[
            jax.ShapeDtypeStruct((N, D), jnp.float32),
            jax.ShapeDtypeStruct((N, 2 * D), jnp.float32),
            jax.ShapeDtypeStruct((N, D), jnp.float32),
        ],
    )(h, Wq, bq.reshape(1, D), Wkv, bkv.reshape(1, 2 * D),
      Ws, bs.reshape(1, D))


def _logits_body(qd_ref, ks_ref, sel_ref, lg_ref, mx_ref):
    i = pl.program_id(0)
    prod = qd_ref[...] * ks_ref[...]
    lg = jnp.dot(prod, sel_ref[...], preferred_element_type=jnp.float32)
    lg_ref[...] = lg
    bmx = jnp.max(lg, axis=0, keepdims=True)

    @pl.when(i == 0)
    def _():
        mx_ref[...] = bmx

    @pl.when(i > 0)
    def _():
        mx_ref[...] = jnp.maximum(mx_ref[...], bmx)


def _logits(qd, kvs, sel):
    return pl.pallas_call(
        _logits_body,
        grid=(E // _BE,),
        in_specs=[
            pl.BlockSpec((_BE, D), lambda i: (i, 0)),
            pl.BlockSpec((_BE, D), lambda i: (i, 0)),
            pl.BlockSpec((D, H), lambda i: (0, 0)),
        ],
        out_specs=[
            pl.BlockSpec((_BE, H), lambda i: (i, 0)),
            pl.BlockSpec((1, H), lambda i: (0, 0)),
        ],
        out_shape=[
            jax.ShapeDtypeStruct((E, H), jnp.float32),
            jax.ShapeDtypeStruct((1, H), jnp.float32),
        ],
    )(qd, kvs, sel)


def _weight_body(lg_ref, sh_ref, vs_ref, exp_ref, ex_ref, w_ref):
    ex = jnp.exp(lg_ref[...] - sh_ref[...])
    ex_ref[:, :H] = ex
    ex_ref[:, H:] = jnp.zeros_like(ex)
    w_ref[...] = vs_ref[...] * jnp.dot(
        ex, exp_ref[...], preferred_element_type=jnp.float32
    )


def _weight(lg, shift, kvs, expand):
    return pl.pallas_call(
        _weight_body,
        grid=(E // _BE,),
        in_specs=[
            pl.BlockSpec((_BE, H), lambda i: (i, 0)),
            pl.BlockSpec((1, H), lambda i: (0, 0)),
            pl.BlockSpec((_BE, D), lambda i: (i, 1)),
            pl.BlockSpec((H, D), lambda i: (0, 0)),
        ],
        out_specs=[
            pl.BlockSpec((_BE, 16), lambda i: (i, 0)),
            pl.BlockSpec((_BE, D), lambda i: (i, 0)),
        ],
        out_shape=[
            jax.ShapeDtypeStruct((E, 16), jnp.float32),
            jax.ShapeDtypeStruct((E, D), jnp.float32),
        ],
    )(lg, shift, kvs, expand)


def _finish_body(p_ref, d_ref, s_ref, exp_ref, o_ref):
    den = d_ref[...][:, :H] + 1e-16
    r = jnp.dot(1.0 / den, exp_ref[...], preferred_element_type=jnp.float32)
    o_ref[...] = jax.nn.relu(p_ref[...] * r + s_ref[...])


def _finish(p, d, s, expand):
    return pl.pallas_call(
        _finish_body,
        grid=(N // _BM,),
        in_specs=[
            pl.BlockSpec((_BM, D), lambda i: (i, 0)),
            pl.BlockSpec((_BM, 16), lambda i: (i, 0)),
            pl.BlockSpec((_BM, D), lambda i: (i, 0)),
            pl.BlockSpec((H, D), lambda i: (0, 0)),
        ],
        out_specs=pl.BlockSpec((_BM, D), lambda i: (i, 0)),
        out_shape=jax.ShapeDtypeStruct((N, D), jnp.float32),
    )(p, d, s, expand)


def kernel(x, edge_index, params):
    dstf = edge_index[1]
    dst3 = edge_index[1].reshape(NW, NJ, CH)
    src3 = edge_index[0].reshape(NW, NJ, CH)
    z16 = jnp.zeros((NPS, 16), jnp.float32)
    z128 = jnp.zeros((NPS, D), jnp.float32)
    heads = np.arange(D) // C
    mask = (heads[:, None] == np.arange(H)[None, :])
    sel = jnp.asarray(mask / np.sqrt(C), jnp.float32)
    expand = jnp.asarray(mask.T.astype(np.float32))

    h = _linear(x, params['W_in'], params['b_in'])
    for l in range(L):
        Wkv = jnp.concatenate(
            [params['Wk%d' % l], params['Wv%d' % l]], axis=1
        )
        bkv = jnp.concatenate([params['bk%d' % l], params['bv%d' % l]])
        q, kv, s = _proj3(h, params['Wq%d' % l], params['bq%d' % l],
                          Wkv, bkv, params['Ws%d' % l], params['bs%d' % l])
        qd, kvs = _sc_gather(q, kv, dst3, src3)
        lg, mx = _logits(qd, kvs, sel)
        shift = jnp.maximum(mx - 50.0, 0.0)
        ex16, w = _weight(lg, shift, kvs, expand)
        den = jax.ops.segment_sum(ex16, dstf, num_segments=N)
        out = jax.ops.segment_sum(w, dstf, num_segments=N)
        h = _finish(out, den, s, expand)
    return _linear(h, params['W_out'], params['b_out'])
